# baseline (device time: 13529 ns/iter reference)
import jax
import jax.numpy as jnp
from jax import lax
from jax.experimental import pallas as pl
from jax.experimental.pallas import tpu as pltpu

N_DEV = 16
N_COL = 2


def kernel(x):
    m_per, n = x.shape
    cw = n // N_COL

    def body(x_ref, out_ref, mine_ref, comm_ref, send_sems, recv_sems):
        my_pos = lax.axis_index("i")
        c = pl.program_id(0)
        barrier_sem = pltpu.get_barrier_semaphore()

        @pl.when(c == 0)
        def _():
            for d in range(1, N_DEV):
                t = lax.rem(my_pos + d, N_DEV)
                pl.semaphore_signal(
                    barrier_sem,
                    inc=1,
                    device_id=(t,),
                    device_id_type=pl.DeviceIdType.MESH,
                )

        partial = jnp.sum(x_ref[:, :], axis=0, keepdims=True)

        for cc in range(N_COL):
            @pl.when(c == cc)
            def _(cc=cc):
                mine_ref[:, cc * cw:(cc + 1) * cw] = partial
                if cc == 0:
                    pl.semaphore_wait(barrier_sem, N_DEV - 1)
                for d in range(1, N_DEV):
                    t = lax.rem(my_pos + d, N_DEV)
                    k = N_DEV - 1 - d
                    rdma = pltpu.make_async_remote_copy(
                        src_ref=mine_ref.at[:, cc * cw:(cc + 1) * cw],
                        dst_ref=comm_ref.at[cc, k],
                        send_sem=send_sems.at[cc, d - 1],
                        recv_sem=recv_sems.at[cc, k],
                        device_id=(t,),
                        device_id_type=pl.DeviceIdType.MESH,
                    )
                    rdma.start()

        @pl.when(c == N_COL - 1)
        def _():
            for cc in range(N_COL):
                for k in range(N_DEV - 1):
                    recv = pltpu.make_async_remote_copy(
                        src_ref=mine_ref.at[:, cc * cw:(cc + 1) * cw],
                        dst_ref=comm_ref.at[cc, k],
                        send_sem=send_sems.at[cc, 0],
                        recv_sem=recv_sems.at[cc, k],
                        device_id=(my_pos,),
                        device_id_type=pl.DeviceIdType.MESH,
                    )
                    recv.wait_recv()
                out_ref[:, cc * cw:(cc + 1) * cw] = mine_ref[
                    :, cc * cw:(cc + 1) * cw
                ] + jnp.sum(comm_ref[cc], axis=0)
            for cc in range(N_COL):
                for d in range(1, N_DEV):
                    send = pltpu.make_async_remote_copy(
                        src_ref=mine_ref.at[:, cc * cw:(cc + 1) * cw],
                        dst_ref=comm_ref.at[cc, 0],
                        send_sem=send_sems.at[cc, d - 1],
                        recv_sem=recv_sems.at[cc, 0],
                        device_id=(my_pos,),
                        device_id_type=pl.DeviceIdType.MESH,
                    )
                    send.wait_send()

    return pl.pallas_call(
        body,
        grid=(N_COL,),
        out_shape=jax.ShapeDtypeStruct((1, n), jnp.float32),
        in_specs=[pl.BlockSpec((m_per, cw), lambda c: (0, c))],
        out_specs=pl.BlockSpec((1, n), lambda c: (0, 0)),
        scratch_shapes=[
            pltpu.VMEM((1, n), jnp.float32),
            pltpu.VMEM((N_COL, N_DEV - 1, 1, cw), jnp.float32),
            pltpu.SemaphoreType.DMA((N_COL, N_DEV - 1)),
            pltpu.SemaphoreType.DMA((N_COL, N_DEV - 1)),
        ],
        compiler_params=pltpu.CompilerParams(collective_id=0),
    )(x)


# device time: 12851 ns/iter; 1.0528x vs baseline; 1.0528x over previous
import jax
import jax.numpy as jnp
from jax import lax
from jax.experimental import pallas as pl
from jax.experimental.pallas import tpu as pltpu

N_DEV = 16
N_STEPS = 4


def kernel(x):
    m_per, n = x.shape
    m_chunk = m_per // N_STEPS

    def body(x_ref, out_ref, mine_ref, acc_ref, comm_ref, send_sems, recv_sems):
        my_pos = lax.axis_index("i")
        step = pl.program_id(0)
        barrier_sem = pltpu.get_barrier_semaphore()

        @pl.when(step == 0)
        def _():
            for d in range(1, N_DEV):
                t = lax.rem(my_pos + d, N_DEV)
                pl.semaphore_signal(
                    barrier_sem,
                    inc=1,
                    device_id=(t,),
                    device_id_type=pl.DeviceIdType.MESH,
                )

        ones = jnp.full((8, m_chunk), 0.125, jnp.float32)
        partial = jax.lax.dot(
            ones, x_ref[:, :], preferred_element_type=jnp.float32
        )

        @pl.when(step == 0)
        def _():
            acc_ref[:, :] = partial

        @pl.when(step > 0)
        def _():
            acc_ref[:, :] = acc_ref[:, :] + partial

        @pl.when(step == N_STEPS - 1)
        def _():
            mine_ref[:, :] = jnp.sum(acc_ref[:, :], axis=0, keepdims=True)

            pl.semaphore_wait(barrier_sem, N_DEV - 1)

            sends = []
            for d in range(1, N_DEV):
                t = lax.rem(my_pos + d, N_DEV)
                rdma = pltpu.make_async_remote_copy(
                    src_ref=mine_ref,
                    dst_ref=comm_ref.at[N_DEV - 1 - d],
                    send_sem=send_sems.at[d - 1],
                    recv_sem=recv_sems.at[N_DEV - 1 - d],
                    device_id=(t,),
                    device_id_type=pl.DeviceIdType.MESH,
                )
                rdma.start()
                sends.append(rdma)

            for k in range(N_DEV - 1):
                recv = pltpu.make_async_remote_copy(
                    src_ref=mine_ref,
                    dst_ref=comm_ref.at[k],
                    send_sem=send_sems.at[0],
                    recv_sem=recv_sems.at[k],
                    device_id=(my_pos,),
                    device_id_type=pl.DeviceIdType.MESH,
                )
                recv.wait_recv()

            out_ref[:, :] = mine_ref[:, :] + jnp.sum(comm_ref[:, :, :], axis=0)

            for rdma in sends:
                rdma.wait_send()

    return pl.pallas_call(
        body,
        grid=(N_STEPS,),
        out_shape=jax.ShapeDtypeStruct((1, n), jnp.float32),
        in_specs=[pl.BlockSpec((m_chunk, n), lambda i: (i, 0))],
        out_specs=pl.BlockSpec((1, n), lambda i: (0, 0)),
        scratch_shapes=[
            pltpu.VMEM((1, n), jnp.float32),
            pltpu.VMEM((8, n), jnp.float32),
            pltpu.VMEM((N_DEV - 1, 1, n), jnp.float32),
            pltpu.SemaphoreType.DMA((N_DEV - 1,)),
            pltpu.SemaphoreType.DMA((N_DEV - 1,)),
        ],
        compiler_params=pltpu.CompilerParams(collective_id=0),
    )(x)


# device time: 12698 ns/iter; 1.0654x vs baseline; 1.0120x over previous
import jax
import jax.numpy as jnp
from jax import lax
from jax.experimental import pallas as pl
from jax.experimental.pallas import tpu as pltpu

N_DEV = 16
N_STEPS = 16


def kernel(x):
    m_per, n = x.shape
    m_chunk = m_per // N_STEPS

    def body(x_ref, out_ref, mine_ref, comm_ref, send_sems, recv_sems):
        my_pos = lax.axis_index("i")
        step = pl.program_id(0)
        barrier_sem = pltpu.get_barrier_semaphore()

        @pl.when(step == 0)
        def _():
            for d in range(1, N_DEV):
                t = lax.rem(my_pos + d, N_DEV)
                pl.semaphore_signal(
                    barrier_sem,
                    inc=1,
                    device_id=(t,),
                    device_id_type=pl.DeviceIdType.MESH,
                )

        partial = jnp.sum(x_ref[:, :], axis=0, keepdims=True)

        @pl.when(step == 0)
        def _():
            mine_ref[:, :] = partial

        @pl.when(step > 0)
        def _():
            mine_ref[:, :] = mine_ref[:, :] + partial

        @pl.when(step == N_STEPS - 1)
        def _():
            pl.semaphore_wait(barrier_sem, N_DEV - 1)

            sends = []
            for d in range(1, N_DEV):
                t = lax.rem(my_pos + d, N_DEV)
                rdma = pltpu.make_async_remote_copy(
                    src_ref=mine_ref,
                    dst_ref=comm_ref.at[N_DEV - 1 - d],
                    send_sem=send_sems.at[d - 1],
                    recv_sem=recv_sems.at[N_DEV - 1 - d],
                    device_id=(t,),
                    device_id_type=pl.DeviceIdType.MESH,
                )
                rdma.start()
                sends.append(rdma)

            for k in range(N_DEV - 1):
                recv = pltpu.make_async_remote_copy(
                    src_ref=mine_ref,
                    dst_ref=comm_ref.at[k],
                    send_sem=send_sems.at[0],
                    recv_sem=recv_sems.at[k],
                    device_id=(my_pos,),
                    device_id_type=pl.DeviceIdType.MESH,
                )
                recv.wait_recv()

            out_ref[:, :] = mine_ref[:, :] + jnp.sum(comm_ref[:, :, :], axis=0)

            for rdma in sends:
                rdma.wait_send()

    return pl.pallas_call(
        body,
        grid=(N_STEPS,),
        out_shape=jax.ShapeDtypeStruct((1, n), jnp.float32),
        in_specs=[pl.BlockSpec((m_chunk, n), lambda i: (i, 0))],
        out_specs=pl.BlockSpec((1, n), lambda i: (0, 0)),
        scratch_shapes=[
            pltpu.VMEM((1, n), jnp.float32),
            pltpu.VMEM((N_DEV - 1, 1, n), jnp.float32),
            pltpu.SemaphoreType.DMA((N_DEV - 1,)),
            pltpu.SemaphoreType.DMA((N_DEV - 1,)),
        ],
        compiler_params=pltpu.CompilerParams(collective_id=0),
    )(x)


# device time: 12447 ns/iter; 1.0869x vs baseline; 1.0202x over previous
import jax
import jax.numpy as jnp
from jax import lax
from jax.experimental import pallas as pl
from jax.experimental.pallas import tpu as pltpu

N_DEV = 16
N_STEPS = 8


def kernel(x):
    m_per, n = x.shape
    m_chunk = m_per // N_STEPS

    def body(x_ref, out_ref, mine_ref, comm_ref, send_sems, recv_sems):
        my_pos = lax.axis_index("i")
        step = pl.program_id(0)
        barrier_sem = pltpu.get_barrier_semaphore()

        @pl.when(step == 0)
        def _():
            for d in range(1, N_DEV):
                t = lax.rem(my_pos + d, N_DEV)
                pl.semaphore_signal(
                    barrier_sem,
                    inc=1,
                    device_id=(t,),
                    device_id_type=pl.DeviceIdType.MESH,
                )

        partial = jnp.sum(x_ref[:, :], axis=0, keepdims=True)

        @pl.when(step == 0)
        def _():
            mine_ref[:, :] = partial

        @pl.when(step > 0)
        def _():
            mine_ref[:, :] = mine_ref[:, :] + partial

        @pl.when(step == N_STEPS - 1)
        def _():
            pl.semaphore_wait(barrier_sem, N_DEV - 1)

            sends = []
            for d in range(1, N_DEV):
                t = lax.rem(my_pos + d, N_DEV)
                rdma = pltpu.make_async_remote_copy(
                    src_ref=mine_ref,
                    dst_ref=comm_ref.at[N_DEV - 1 - d],
                    send_sem=send_sems.at[d - 1],
                    recv_sem=recv_sems.at[N_DEV - 1 - d],
                    device_id=(t,),
                    device_id_type=pl.DeviceIdType.MESH,
                )
                rdma.start()
                sends.append(rdma)

            for k in range(N_DEV - 1):
                recv = pltpu.make_async_remote_copy(
                    src_ref=mine_ref,
                    dst_ref=comm_ref.at[k],
                    send_sem=send_sems.at[0],
                    recv_sem=recv_sems.at[k],
                    device_id=(my_pos,),
                    device_id_type=pl.DeviceIdType.MESH,
                )
                recv.wait_recv()

            out_ref[:, :] = mine_ref[:, :] + jnp.sum(comm_ref[:, :, :], axis=0)

            for rdma in sends:
                rdma.wait_send()

    return pl.pallas_call(
        body,
        grid=(N_STEPS,),
        out_shape=jax.ShapeDtypeStruct((1, n), jnp.float32),
        in_specs=[pl.BlockSpec((m_chunk, n), lambda i: (i, 0))],
        out_specs=pl.BlockSpec((1, n), lambda i: (0, 0)),
        scratch_shapes=[
            pltpu.VMEM((1, n), jnp.float32),
            pltpu.VMEM((N_DEV - 1, 1, n), jnp.float32),
            pltpu.SemaphoreType.DMA((N_DEV - 1,)),
            pltpu.SemaphoreType.DMA((N_DEV - 1,)),
        ],
        compiler_params=pltpu.CompilerParams(collective_id=0),
    )(x)
